# Initial kernel scaffold; baseline (speedup 1.0000x reference)
import jax
import jax.numpy as jnp
from jax.experimental import pallas as pl


def kernel(patch_features):
    raise NotImplementedError("write your pallas kernel here")


# speedup vs baseline: 11.2002x; 11.2002x over previous
import jax
import jax.numpy as jnp
import numpy as np
from jax import lax
from jax.experimental import pallas as pl
from jax.experimental.pallas import tpu as pltpu

_B, _N, _C = 32, 1024, 96
_CLUSTER = 256
_K = 5
_SQRT_C = _C ** 0.5
_INV_K = np.float32(1.0 / _K)


def _dpc_body(x_ref, x2r_ref, x2c_ref, noise_ref, eye_ref, idx_ref, mrg_ref):
    f32 = jnp.float32
    x = x_ref[0]
    x2r = x2r_ref[0]
    x2c = x2c_ref[0]
    noise = noise_ref[0]
    eye = eye_ref[...]

    g = lax.dot_general(x, x, (((1,), (1,)), ((), ())))
    d2 = (x2c + x2r) - 2.0 * g
    d2 = jnp.maximum(d2, 0.0)
    dist = jnp.sqrt(d2) / _SQRT_C

    dw = dist
    rem = jnp.full((1, _N), float(_K), f32)
    ms, poss, takes = [], [], []
    for _ in range(_K):
        m = jnp.min(dw, axis=0, keepdims=True)
        eq = dw == m
        cnt = jnp.sum(eq.astype(f32), axis=0, keepdims=True)
        take = jnp.minimum(cnt, rem)
        ms.append(m * m)
        poss.append(float(_K) - rem)
        takes.append(take)
        rem = rem - take
        dw = jnp.where(eq, jnp.inf, dw)
    s = []
    for j in range(_K):
        sj = jnp.zeros((1, _N), f32)
        for t in range(_K):
            hit = (poss[t] <= float(j)) & (poss[t] + takes[t] > float(j))
            sj = sj + jnp.where(hit, ms[t], 0.0)
        s.append(sj)
    acc = ((s[0] + s[4]) + s[2]) + (s[1] + s[3])
    dens_row = jnp.exp(-(acc * _INV_K)) + noise
    dens_col = jnp.sum(eye * dens_row, axis=1, keepdims=True)

    dmax = jnp.max(dist)
    masked = jnp.where(dens_col > dens_row, dist, dmax)
    ind_row = jnp.min(masked, axis=0, keepdims=True)
    score_row = ind_row * dens_row
    score_col = jnp.sum(eye * score_row, axis=1, keepdims=True)

    jlt = lax.broadcasted_iota(jnp.int32, (_N, _N), 0) < lax.broadcasted_iota(
        jnp.int32, (_N, _N), 1)
    cmp = (score_col > score_row) | ((score_col == score_row) & jlt)
    rank_row = jnp.sum(cmp.astype(f32), axis=0, keepdims=True)

    cio = lax.broadcasted_iota(jnp.int32, (_CLUSTER, _N), 0).astype(f32)
    nio = lax.broadcasted_iota(jnp.int32, (_CLUSTER, _N), 1).astype(f32)
    sel = (cio == jnp.broadcast_to(rank_row, (_CLUSTER, _N))).astype(f32)
    idx_down = jnp.sum(sel * nio, axis=1, keepdims=True)

    dm = lax.dot_general(sel, dist, (((1,), (0,)), ((), ())),
                         precision=lax.Precision.HIGHEST)
    minv = jnp.min(dm, axis=0, keepdims=True)
    cands = jnp.where(dm == minv, cio, 65536.0)
    cl_row = jnp.min(cands, axis=0, keepdims=True)
    is_center = jnp.sum(sel, axis=0, keepdims=True)
    center_c = jnp.sum(sel * cio, axis=0, keepdims=True)
    cl_row = jnp.where(is_center > 0.5, center_c, cl_row)

    assign = (cio == jnp.broadcast_to(cl_row, (_CLUSTER, _N))).astype(f32)
    counts = jnp.sum(assign, axis=1, keepdims=True)
    invw = 1.0 / (counts + 1e-6)
    norm_row = jnp.sum(assign * invw, axis=0, keepdims=True)
    aw = assign * norm_row
    merged = lax.dot_general(aw, x, (((1,), (0,)), ((), ())),
                             precision=lax.Precision.HIGHEST)

    idx_ref[0] = idx_down.astype(jnp.int32)
    mrg_ref[0] = merged


def kernel(patch_features):
    x = patch_features
    x2 = jnp.sum(x * x, axis=-1)
    noise = jax.random.uniform(jax.random.key(1), (_B, _N),
                               dtype=jnp.float32) * 1e-6
    eye = jnp.eye(_N, dtype=jnp.float32)

    idx3, merged = pl.pallas_call(
        _dpc_body,
        grid=(_B,),
        in_specs=[
            pl.BlockSpec((1, _N, _C), lambda b: (b, 0, 0)),
            pl.BlockSpec((1, 1, _N), lambda b: (b, 0, 0)),
            pl.BlockSpec((1, _N, 1), lambda b: (b, 0, 0)),
            pl.BlockSpec((1, 1, _N), lambda b: (b, 0, 0)),
            pl.BlockSpec((_N, _N), lambda b: (0, 0)),
        ],
        out_specs=[
            pl.BlockSpec((1, _CLUSTER, 1), lambda b: (b, 0, 0)),
            pl.BlockSpec((1, _CLUSTER, _C), lambda b: (b, 0, 0)),
        ],
        out_shape=[
            jax.ShapeDtypeStruct((_B, _CLUSTER, 1), jnp.int32),
            jax.ShapeDtypeStruct((_B, _CLUSTER, _C), jnp.float32),
        ],
        compiler_params=pltpu.CompilerParams(
            dimension_semantics=("parallel",)),
    )(x, x2[:, None, :], x2[:, :, None], noise[:, None, :], eye)

    return (idx3[:, :, 0], merged)


# speedup vs baseline: 14.2294x; 1.2705x over previous
import jax
import jax.numpy as jnp
import numpy as np
from jax import lax
from jax.experimental import pallas as pl
from jax.experimental.pallas import tpu as pltpu

_B, _N, _C = 32, 1024, 96
_CLUSTER = 256
_K = 5
_SQRT_C = _C ** 0.5
_INV_K = np.float32(1.0 / _K)


def _dpc_body(x_ref, x2r_ref, x2c_ref, noise_ref, eye_ref, idx_ref, mrg_ref):
    f32 = jnp.float32
    x = x_ref[0]
    x2r = x2r_ref[0]
    x2c = x2c_ref[0]
    noise = noise_ref[0]
    eye = eye_ref[...]

    g = lax.dot_general(x, x, (((1,), (1,)), ((), ())))
    d2 = (x2c + x2r) - 2.0 * g
    d2 = jnp.maximum(d2, 0.0)
    dist = jnp.sqrt(d2) / _SQRT_C

    dw = dist
    rem = jnp.full((1, _N), float(_K), f32)
    ms, poss, takes = [], [], []
    for _ in range(_K):
        m = jnp.min(dw, axis=0, keepdims=True)
        eq = dw == m
        cnt = jnp.sum(eq.astype(f32), axis=0, keepdims=True)
        take = jnp.minimum(cnt, rem)
        ms.append(m * m)
        poss.append(float(_K) - rem)
        takes.append(take)
        rem = rem - take
        dw = jnp.where(eq, jnp.inf, dw)
    s = []
    for j in range(_K):
        sj = jnp.zeros((1, _N), f32)
        for t in range(_K):
            hit = (poss[t] <= float(j)) & (poss[t] + takes[t] > float(j))
            sj = sj + jnp.where(hit, ms[t], 0.0)
        s.append(sj)
    acc = ((s[0] + s[4]) + s[2]) + (s[1] + s[3])
    dens_row = jnp.exp(-(acc * _INV_K)) + noise
    dens_col = jnp.sum(eye * dens_row, axis=1, keepdims=True)

    dmax = jnp.max(dist)
    masked = jnp.where(dens_col > dens_row, dist, dmax)
    ind_row = jnp.min(masked, axis=0, keepdims=True)
    score_row = ind_row * dens_row
    score_col = jnp.sum(eye * score_row, axis=1, keepdims=True)

    jlt = lax.broadcasted_iota(jnp.int32, (_N, _N), 0) < lax.broadcasted_iota(
        jnp.int32, (_N, _N), 1)
    cmp = (score_col > score_row) | ((score_col == score_row) & jlt)
    rank_row = jnp.sum(cmp.astype(f32), axis=0, keepdims=True)

    cio = lax.broadcasted_iota(jnp.int32, (_CLUSTER, _N), 0).astype(f32)
    nio = lax.broadcasted_iota(jnp.int32, (_CLUSTER, _N), 1).astype(f32)
    sel = (cio == jnp.broadcast_to(rank_row, (_CLUSTER, _N))).astype(f32)
    idx_down = jnp.sum(sel * nio, axis=1, keepdims=True)

    rank_col = jnp.sum(eye * rank_row, axis=1, keepdims=True)
    is_center_col = rank_col < float(_CLUSTER)
    bigd = jnp.where(is_center_col, dist, 1e9)
    dmin_row = jnp.min(bigd, axis=0, keepdims=True)
    candr = jnp.where(bigd == dmin_row, rank_col, 65536.0)
    cl_row = jnp.min(candr, axis=0, keepdims=True)
    cl_row = jnp.where(rank_row < float(_CLUSTER), rank_row, cl_row)

    assign = (cio == jnp.broadcast_to(cl_row, (_CLUSTER, _N))).astype(f32)
    counts = jnp.sum(assign, axis=1, keepdims=True)
    invw = 1.0 / (counts + 1e-6)
    norm_row = jnp.sum(assign * invw, axis=0, keepdims=True)
    aw = assign * norm_row
    merged = lax.dot_general(aw, x, (((1,), (0,)), ((), ())),
                             precision=lax.Precision.HIGHEST)

    idx_ref[0] = idx_down.astype(jnp.int32)
    mrg_ref[0] = merged


def kernel(patch_features):
    x = patch_features
    x2 = jnp.sum(x * x, axis=-1)
    noise = jax.random.uniform(jax.random.key(1), (_B, _N),
                               dtype=jnp.float32) * 1e-6
    eye = jnp.eye(_N, dtype=jnp.float32)

    idx3, merged = pl.pallas_call(
        _dpc_body,
        grid=(_B,),
        in_specs=[
            pl.BlockSpec((1, _N, _C), lambda b: (b, 0, 0)),
            pl.BlockSpec((1, 1, _N), lambda b: (b, 0, 0)),
            pl.BlockSpec((1, _N, 1), lambda b: (b, 0, 0)),
            pl.BlockSpec((1, 1, _N), lambda b: (b, 0, 0)),
            pl.BlockSpec((_N, _N), lambda b: (0, 0)),
        ],
        out_specs=[
            pl.BlockSpec((1, _CLUSTER, 1), lambda b: (b, 0, 0)),
            pl.BlockSpec((1, _CLUSTER, _C), lambda b: (b, 0, 0)),
        ],
        out_shape=[
            jax.ShapeDtypeStruct((_B, _CLUSTER, 1), jnp.int32),
            jax.ShapeDtypeStruct((_B, _CLUSTER, _C), jnp.float32),
        ],
        compiler_params=pltpu.CompilerParams(
            dimension_semantics=("parallel",)),
    )(x, x2[:, None, :], x2[:, :, None], noise[:, None, :], eye)

    return (idx3[:, :, 0], merged)
